# baseline (speedup 1.0000x reference)
import jax
import jax.numpy as jnp
from jax.experimental import pallas as pl

_K = 1000
_KP = 1024
_D = 80
_TB = 1024
_KC = 256


def _prep_kernel(cb_ref, cbn_ref):
    cb = cb_ref[...]
    n = jnp.sqrt(jnp.sum(cb * cb, axis=-1, keepdims=True))
    cbn = (cb / jnp.clip(n, 1e-12, None)).astype(jnp.bfloat16)
    cbn_ref[...] = cbn.T


def _label_kernel(x_ref, rp_ref, cbn_ref, o_ref):
    x = x_ref[...]
    mu = jnp.mean(x, axis=-1, keepdims=True)
    xc = x - mu
    var = jnp.mean(xc * xc, axis=-1, keepdims=True)
    xn = xc / jnp.sqrt(var + 1e-5)
    proj = jax.lax.dot_general(
        xn.astype(jnp.bfloat16), rp_ref[...],
        (((1,), (0,)), ((), ())), preferred_element_type=jnp.float32)
    pn = (proj / jnp.clip(
        jnp.sqrt(jnp.sum(proj * proj, axis=-1, keepdims=True)), 1e-12, None)
    ).astype(jnp.bfloat16)
    run_m = None
    run_i = None
    for c in range(_KP // _KC):
        sim = jax.lax.dot_general(
            pn, cbn_ref[:, c * _KC:(c + 1) * _KC],
            (((1,), (0,)), ((), ())), preferred_element_type=jnp.float32)
        if (c + 1) * _KC > _K:
            col = jax.lax.broadcasted_iota(jnp.int32, sim.shape, 1)
            sim = jnp.where(col + c * _KC < _K, sim, -jnp.inf)
        m = jnp.max(sim, axis=-1)
        i = jnp.argmax(sim, axis=-1).astype(jnp.int32) + c * _KC
        if run_m is None:
            run_m, run_i = m, i
        else:
            take = m > run_m
            run_i = jnp.where(take, i, run_i)
            run_m = jnp.maximum(run_m, m)
    o_ref[0, 0, :] = run_i


def kernel(features, rand_proj, codebook):
    B, T, D = features.shape
    K, E = codebook.shape
    cb_pad = jnp.pad(codebook, ((0, _KP - K), (0, 0)))
    cbn_t = pl.pallas_call(
        _prep_kernel,
        out_shape=jax.ShapeDtypeStruct((E, _KP), jnp.bfloat16),
    )(cb_pad)
    rp_bf = rand_proj.astype(jnp.bfloat16)
    N = B * T
    nb = N // _TB
    xf = features.reshape(N, D)
    out = pl.pallas_call(
        _label_kernel,
        grid=(nb,),
        in_specs=[pl.BlockSpec((_TB, D), lambda i: (i, 0)),
                  pl.BlockSpec((D, E), lambda i: (0, 0)),
                  pl.BlockSpec((E, _KP), lambda i: (0, 0))],
        out_specs=pl.BlockSpec((1, 1, _TB), lambda i: (i, 0, 0)),
        out_shape=jax.ShapeDtypeStruct((nb, 1, _TB), jnp.int32),
    )(xf, rp_bf, cbn_t)
    return out.reshape(B, T)


# speedup vs baseline: 3.5175x; 3.5175x over previous
import jax
import jax.numpy as jnp
from jax.experimental import pallas as pl

_K = 1000
_KP = 1024
_D = 80
_TB = 1024


def _prep_kernel(cb_ref, cbn_ref):
    cb = cb_ref[...]
    n = jnp.sqrt(jnp.sum(cb * cb, axis=-1, keepdims=True))
    cbn_ref[...] = (cb / jnp.clip(n, 1e-12, None)).astype(jnp.bfloat16)


def _label_kernel(xt_ref, rpt_ref, cbn_ref, o_ref):
    xt = xt_ref[...]
    mu = jnp.mean(xt, axis=0, keepdims=True)
    xc = xt - mu
    var = jnp.mean(xc * xc, axis=0, keepdims=True)
    xn = xc / jnp.sqrt(var + 1e-5)
    projt = jax.lax.dot_general(
        rpt_ref[...], xn.astype(jnp.bfloat16),
        (((1,), (0,)), ((), ())), preferred_element_type=jnp.float32)
    pnt = (projt / jnp.clip(
        jnp.sqrt(jnp.sum(projt * projt, axis=0, keepdims=True)), 1e-12, None)
    ).astype(jnp.bfloat16)
    simt = jax.lax.dot_general(
        cbn_ref[...], pnt,
        (((1,), (0,)), ((), ())), preferred_element_type=jnp.float32)
    row = jax.lax.broadcasted_iota(jnp.int32, simt.shape, 0)
    simt = jnp.where(row < _K, simt, -jnp.inf)
    o_ref[0, 0, :] = jnp.argmax(simt, axis=0).astype(jnp.int32)


def kernel(features, rand_proj, codebook):
    B, T, D = features.shape
    K, E = codebook.shape
    cb_pad = jnp.pad(codebook, ((0, _KP - K), (0, 0)))
    cbn = pl.pallas_call(
        _prep_kernel,
        out_shape=jax.ShapeDtypeStruct((_KP, E), jnp.bfloat16),
    )(cb_pad)
    rpt_bf = rand_proj.T.astype(jnp.bfloat16)
    N = B * T
    nb = N // _TB
    xt = features.reshape(N, D).T
    out = pl.pallas_call(
        _label_kernel,
        grid=(nb,),
        in_specs=[pl.BlockSpec((D, _TB), lambda i: (0, i)),
                  pl.BlockSpec((E, D), lambda i: (0, 0)),
                  pl.BlockSpec((_KP, E), lambda i: (0, 0))],
        out_specs=pl.BlockSpec((1, 1, _TB), lambda i: (i, 0, 0)),
        out_shape=jax.ShapeDtypeStruct((nb, 1, _TB), jnp.int32),
    )(xt, rpt_bf, cbn)
    return out.reshape(B, T)
